# Initial kernel scaffold; baseline (speedup 1.0000x reference)
import jax
import jax.numpy as jnp
from jax.experimental import pallas as pl


def kernel(feature, W1, W2, gamma, beta):
    raise NotImplementedError("write your pallas kernel here")


# speedup vs baseline: 13.0355x; 13.0355x over previous
import functools

import jax
import jax.numpy as jnp
from jax import lax
from jax.experimental import pallas as pl
from jax.experimental.pallas import tpu as pltpu
from jax.experimental.pallas import tpu_sc as plsc

B = 4
C = 64
N = 4096
K = 16

T_KNN = 256
T_ST = 128
T_FIN = 512
CP = 128


def _knn_conv_body(ft_ref, f_ref, w1t_ref, w2t_ref, knn_ref, s_ref, et_ref):
    b = pl.program_id(0)
    ftT = ft_ref[0]
    fb = f_ref[0]
    w1t = w1t_ref[...]
    w2t = w2t_ref[...]

    et = jnp.dot(ftT, w2t, preferred_element_type=jnp.float32)
    s = jnp.dot(ftT, w1t + w2t, preferred_element_type=jnp.float32)
    et_ref[0, :, :C] = et
    et_ref[0, :, C:] = jnp.zeros((ftT.shape[0], CP - C), jnp.float32)
    s_ref[0] = s

    sq = jnp.sum(fb * fb, axis=0, keepdims=True)
    g = jnp.dot(ftT, fb, preferred_element_type=jnp.float32)
    score = sq - 2.0 * g
    iota = lax.broadcasted_iota(jnp.int32, score.shape, 1)
    boff = b * N
    for kstep in range(K):
        m = jnp.min(score, axis=1, keepdims=True)
        cand = jnp.where(score == m, iota, N)
        idx = jnp.min(cand, axis=1, keepdims=True)
        knn_ref[0, :, kstep:kstep + 1] = idx + boff
        score = jnp.where(iota == idx, jnp.inf, score)


def _knn_conv(fT, feature, w1t, w2t):
    grid = (B, N // T_KNN)
    return pl.pallas_call(
        _knn_conv_body,
        grid=grid,
        in_specs=[
            pl.BlockSpec((1, T_KNN, C), lambda b, t: (b, t, 0)),
            pl.BlockSpec((1, C, N), lambda b, t: (b, 0, 0)),
            pl.BlockSpec((C, C), lambda b, t: (0, 0)),
            pl.BlockSpec((C, C), lambda b, t: (0, 0)),
        ],
        out_specs=[
            pl.BlockSpec((1, T_KNN, K), lambda b, t: (b, t, 0)),
            pl.BlockSpec((1, T_KNN, C), lambda b, t: (b, t, 0)),
            pl.BlockSpec((1, T_KNN, CP), lambda b, t: (b, t, 0)),
        ],
        out_shape=[
            jax.ShapeDtypeStruct((B, N, K), jnp.int32),
            jax.ShapeDtypeStruct((B, N, C), jnp.float32),
            jax.ShapeDtypeStruct((B, N, CP), jnp.float32),
        ],
    )(fT, feature, w1t, w2t)


def _sc_gather(idx_flat, table):
    R = idx_flat.shape[0]
    D = table.shape[1]
    NW = 32
    Q = R // NW
    CH = 128
    nchunk = Q // CH
    mesh = plsc.VectorSubcoreMesh(core_axis_name="c", subcore_axis_name="s")

    @functools.partial(
        pl.kernel,
        mesh=mesh,
        out_type=jax.ShapeDtypeStruct((R, D), jnp.float32),
        scratch_types=[
            pltpu.VMEM((CH,), jnp.int32),
            pltpu.VMEM((CH, D), jnp.float32),
            pltpu.SemaphoreType.DMA,
        ],
    )
    def gk(idx_hbm, tab_hbm, out_hbm, idx_v, rows_v, sem):
        wid = lax.axis_index("s") * 2 + lax.axis_index("c")

        def chunk(i, carry):
            base = pl.multiple_of(wid * Q + i * CH, CH)
            pltpu.sync_copy(idx_hbm.at[pl.ds(base, CH)], idx_v)
            pltpu.async_copy(tab_hbm.at[idx_v], rows_v, sem).wait()
            pltpu.sync_copy(rows_v, out_hbm.at[pl.ds(base, CH)])
            return carry

        lax.fori_loop(0, nchunk, chunk, 0)

    return gk(idx_flat, table)


def _edge_stats_body(nb_ref, s_ref, mx_ref, mn_ref, sum_ref, ssq_ref):
    first = (pl.program_id(0) == 0) & (pl.program_id(1) == 0)
    s = s_ref[0]
    mx = None
    mn = None
    psum = jnp.zeros((1, C), jnp.float32)
    psq = jnp.zeros((1, C), jnp.float32)
    for k in range(K):
        ef = s - nb_ref[0, :, k, :C]
        mx = ef if mx is None else jnp.maximum(mx, ef)
        mn = ef if mn is None else jnp.minimum(mn, ef)
        psum = psum + jnp.sum(ef, axis=0, keepdims=True)
        psq = psq + jnp.sum(ef * ef, axis=0, keepdims=True)
    mx_ref[0] = mx
    mn_ref[0] = mn

    @pl.when(first)
    def _init():
        sum_ref[...] = jnp.zeros_like(sum_ref)
        ssq_ref[...] = jnp.zeros_like(ssq_ref)

    sum_ref[...] += psum
    ssq_ref[...] += psq


def _edge_stats(nb4, S):
    grid = (B, N // T_ST)
    return pl.pallas_call(
        _edge_stats_body,
        grid=grid,
        in_specs=[
            pl.BlockSpec((1, T_ST, K, CP), lambda b, t: (b, t, 0, 0)),
            pl.BlockSpec((1, T_ST, C), lambda b, t: (b, t, 0)),
        ],
        out_specs=[
            pl.BlockSpec((1, T_ST, C), lambda b, t: (b, t, 0)),
            pl.BlockSpec((1, T_ST, C), lambda b, t: (b, t, 0)),
            pl.BlockSpec((1, C), lambda b, t: (0, 0)),
            pl.BlockSpec((1, C), lambda b, t: (0, 0)),
        ],
        out_shape=[
            jax.ShapeDtypeStruct((B, N, C), jnp.float32),
            jax.ShapeDtypeStruct((B, N, C), jnp.float32),
            jax.ShapeDtypeStruct((1, C), jnp.float32),
            jax.ShapeDtypeStruct((1, C), jnp.float32),
        ],
    )(nb4, S)


def _finalize_body(mx_ref, mn_ref, sum_ref, ssq_ref, g_ref, be_ref, out_ref):
    cnt = float(B * N * K)
    mean = sum_ref[...] / cnt
    var = ssq_ref[...] / cnt - mean * mean
    inv = lax.rsqrt(var + 1e-5)
    scale = g_ref[...] * inv
    shift = be_ref[...] - mean * scale
    sel = jnp.where(scale >= 0.0, mx_ref[0], mn_ref[0])
    out_ref[0] = jnp.maximum(sel * scale + shift, 0.0)


def _finalize(mx, mn, ssum, ssq, gamma2, beta2):
    grid = (B, N // T_FIN)
    return pl.pallas_call(
        _finalize_body,
        grid=grid,
        in_specs=[
            pl.BlockSpec((1, T_FIN, C), lambda b, t: (b, t, 0)),
            pl.BlockSpec((1, T_FIN, C), lambda b, t: (b, t, 0)),
            pl.BlockSpec((1, C), lambda b, t: (0, 0)),
            pl.BlockSpec((1, C), lambda b, t: (0, 0)),
            pl.BlockSpec((1, C), lambda b, t: (0, 0)),
            pl.BlockSpec((1, C), lambda b, t: (0, 0)),
        ],
        out_specs=pl.BlockSpec((1, T_FIN, C), lambda b, t: (b, t, 0)),
        out_shape=jax.ShapeDtypeStruct((B, N, C), jnp.float32),
    )(mx, mn, ssum, ssq, gamma2, beta2)


def kernel(feature, W1, W2, gamma, beta):
    fT = feature.transpose(0, 2, 1)
    knn, S, Et = _knn_conv(fT, feature, W1.T, W2.T)
    nb_flat = _sc_gather(knn.reshape(-1), Et.reshape(B * N, CP))
    nb4 = nb_flat.reshape(B, N, K, CP)
    mx, mn, ssum, ssq = _edge_stats(nb4, S)
    out = _finalize(mx, mn, ssum, ssq,
                    gamma.reshape(1, C), beta.reshape(1, C))
    return out.transpose(0, 2, 1)


# speedup vs baseline: 16.4141x; 1.2592x over previous
import functools

import jax
import jax.numpy as jnp
from jax import lax
from jax.experimental import pallas as pl
from jax.experimental.pallas import tpu as pltpu
from jax.experimental.pallas import tpu_sc as plsc

B = 4
C = 64
N = 4096
K = 16

T_KNN = 256
T_ST = 128
T_FIN = 512
CP = 128


def _knn_conv_body(ft_ref, f_ref, w1t_ref, w2t_ref, knn_ref, s_ref, et_ref):
    b = pl.program_id(0)
    ftT = ft_ref[0]
    fb = f_ref[0]
    w1t = w1t_ref[...]
    w2t = w2t_ref[...]

    et = jnp.dot(ftT, w2t, preferred_element_type=jnp.float32)
    s = jnp.dot(ftT, w1t + w2t, preferred_element_type=jnp.float32)
    et_ref[0, :, :C] = et
    et_ref[0, :, C:] = jnp.zeros((ftT.shape[0], CP - C), jnp.float32)
    s_ref[0] = s

    sq = jnp.sum(fb * fb, axis=0, keepdims=True)
    g = jnp.dot(ftT, fb, preferred_element_type=jnp.float32)
    score = sq - 2.0 * g
    iota = lax.broadcasted_iota(jnp.int32, score.shape, 1)
    boff = b * N
    inf = jnp.float32(jnp.inf)
    mprev = None
    for kstep in range(K):
        v = score if mprev is None else jnp.where(score > mprev, score, inf)
        i = iota
        w = N
        while w > 128:
            h = w // 2
            take = v[:, :h] <= v[:, h:w]
            v = jnp.where(take, v[:, :h], v[:, h:w])
            i = jnp.where(take, i[:, :h], i[:, h:w])
            w = h
        m = jnp.min(v, axis=1, keepdims=True)
        idx = jnp.min(jnp.where(v == m, i, N), axis=1, keepdims=True)
        knn_ref[0, :, kstep:kstep + 1] = idx + boff
        mprev = m


def _knn_conv(fT, feature, w1t, w2t):
    grid = (B, N // T_KNN)
    return pl.pallas_call(
        _knn_conv_body,
        grid=grid,
        in_specs=[
            pl.BlockSpec((1, T_KNN, C), lambda b, t: (b, t, 0)),
            pl.BlockSpec((1, C, N), lambda b, t: (b, 0, 0)),
            pl.BlockSpec((C, C), lambda b, t: (0, 0)),
            pl.BlockSpec((C, C), lambda b, t: (0, 0)),
        ],
        out_specs=[
            pl.BlockSpec((1, T_KNN, K), lambda b, t: (b, t, 0)),
            pl.BlockSpec((1, T_KNN, C), lambda b, t: (b, t, 0)),
            pl.BlockSpec((1, T_KNN, CP), lambda b, t: (b, t, 0)),
        ],
        out_shape=[
            jax.ShapeDtypeStruct((B, N, K), jnp.int32),
            jax.ShapeDtypeStruct((B, N, C), jnp.float32),
            jax.ShapeDtypeStruct((B, N, CP), jnp.float32),
        ],
    )(fT, feature, w1t, w2t)


def _sc_gather(idx_flat, table):
    R = idx_flat.shape[0]
    D = table.shape[1]
    NW = 32
    Q = R // NW
    CH = 128
    nchunk = Q // CH
    mesh = plsc.VectorSubcoreMesh(core_axis_name="c", subcore_axis_name="s")

    @functools.partial(
        pl.kernel,
        mesh=mesh,
        out_type=jax.ShapeDtypeStruct((R, D), jnp.float32),
        scratch_types=[
            pltpu.VMEM((CH,), jnp.int32),
            pltpu.VMEM((CH, D), jnp.float32),
            pltpu.SemaphoreType.DMA,
        ],
    )
    def gk(idx_hbm, tab_hbm, out_hbm, idx_v, rows_v, sem):
        wid = lax.axis_index("s") * 2 + lax.axis_index("c")

        def chunk(i, carry):
            base = pl.multiple_of(wid * Q + i * CH, CH)
            pltpu.sync_copy(idx_hbm.at[pl.ds(base, CH)], idx_v)
            pltpu.async_copy(tab_hbm.at[idx_v], rows_v, sem).wait()
            pltpu.sync_copy(rows_v, out_hbm.at[pl.ds(base, CH)])
            return carry

        lax.fori_loop(0, nchunk, chunk, 0)

    return gk(idx_flat, table)


def _edge_stats_body(nb_ref, s_ref, mx_ref, mn_ref, sum_ref, ssq_ref):
    first = (pl.program_id(0) == 0) & (pl.program_id(1) == 0)
    s = s_ref[0]
    mx = None
    mn = None
    psum = jnp.zeros((1, C), jnp.float32)
    psq = jnp.zeros((1, C), jnp.float32)
    for k in range(K):
        ef = s - nb_ref[0, :, k, :C]
        mx = ef if mx is None else jnp.maximum(mx, ef)
        mn = ef if mn is None else jnp.minimum(mn, ef)
        psum = psum + jnp.sum(ef, axis=0, keepdims=True)
        psq = psq + jnp.sum(ef * ef, axis=0, keepdims=True)
    mx_ref[0] = mx
    mn_ref[0] = mn

    @pl.when(first)
    def _init():
        sum_ref[...] = jnp.zeros_like(sum_ref)
        ssq_ref[...] = jnp.zeros_like(ssq_ref)

    sum_ref[...] += psum
    ssq_ref[...] += psq


def _edge_stats(nb4, S):
    grid = (B, N // T_ST)
    return pl.pallas_call(
        _edge_stats_body,
        grid=grid,
        in_specs=[
            pl.BlockSpec((1, T_ST, K, CP), lambda b, t: (b, t, 0, 0)),
            pl.BlockSpec((1, T_ST, C), lambda b, t: (b, t, 0)),
        ],
        out_specs=[
            pl.BlockSpec((1, T_ST, C), lambda b, t: (b, t, 0)),
            pl.BlockSpec((1, T_ST, C), lambda b, t: (b, t, 0)),
            pl.BlockSpec((1, C), lambda b, t: (0, 0)),
            pl.BlockSpec((1, C), lambda b, t: (0, 0)),
        ],
        out_shape=[
            jax.ShapeDtypeStruct((B, N, C), jnp.float32),
            jax.ShapeDtypeStruct((B, N, C), jnp.float32),
            jax.ShapeDtypeStruct((1, C), jnp.float32),
            jax.ShapeDtypeStruct((1, C), jnp.float32),
        ],
    )(nb4, S)


def _finalize_body(mx_ref, mn_ref, sum_ref, ssq_ref, g_ref, be_ref, out_ref):
    cnt = float(B * N * K)
    mean = sum_ref[...] / cnt
    var = ssq_ref[...] / cnt - mean * mean
    inv = lax.rsqrt(var + 1e-5)
    scale = g_ref[...] * inv
    shift = be_ref[...] - mean * scale
    sel = jnp.where(scale >= 0.0, mx_ref[0], mn_ref[0])
    out_ref[0] = jnp.maximum(sel * scale + shift, 0.0)


def _finalize(mx, mn, ssum, ssq, gamma2, beta2):
    grid = (B, N // T_FIN)
    return pl.pallas_call(
        _finalize_body,
        grid=grid,
        in_specs=[
            pl.BlockSpec((1, T_FIN, C), lambda b, t: (b, t, 0)),
            pl.BlockSpec((1, T_FIN, C), lambda b, t: (b, t, 0)),
            pl.BlockSpec((1, C), lambda b, t: (0, 0)),
            pl.BlockSpec((1, C), lambda b, t: (0, 0)),
            pl.BlockSpec((1, C), lambda b, t: (0, 0)),
            pl.BlockSpec((1, C), lambda b, t: (0, 0)),
        ],
        out_specs=pl.BlockSpec((1, T_FIN, C), lambda b, t: (b, t, 0)),
        out_shape=jax.ShapeDtypeStruct((B, N, C), jnp.float32),
    )(mx, mn, ssum, ssq, gamma2, beta2)


def kernel(feature, W1, W2, gamma, beta):
    fT = feature.transpose(0, 2, 1)
    knn, S, Et = _knn_conv(fT, feature, W1.T, W2.T)
    nb_flat = _sc_gather(knn.reshape(-1), Et.reshape(B * N, CP))
    nb4 = nb_flat.reshape(B, N, K, CP)
    mx, mn, ssum, ssq = _edge_stats(nb4, S)
    out = _finalize(mx, mn, ssum, ssq,
                    gamma.reshape(1, C), beta.reshape(1, C))
    return out.transpose(0, 2, 1)


# speedup vs baseline: 16.5974x; 1.0112x over previous
import functools

import jax
import jax.numpy as jnp
from jax import lax
from jax.experimental import pallas as pl
from jax.experimental.pallas import tpu as pltpu
from jax.experimental.pallas import tpu_sc as plsc

B = 4
C = 64
N = 4096
K = 16

T_KNN = 256
T_ST = 128
T_FIN = 512
CP = 128


def _knn_conv_body(fc_ref, f_ref, w1t_ref, w2t_ref, knn_ref, s_ref, et_ref):
    b = pl.program_id(0)
    fc = fc_ref[0]
    fb = f_ref[0]
    w1t = w1t_ref[...]
    w2t = w2t_ref[...]

    cdims = (((0,), (0,)), ((), ()))
    et = lax.dot_general(fc, w2t, cdims, preferred_element_type=jnp.float32)
    s = lax.dot_general(fc, w1t + w2t, cdims, preferred_element_type=jnp.float32)
    et_ref[0, :, :C] = et
    et_ref[0, :, C:] = jnp.zeros((fc.shape[1], CP - C), jnp.float32)
    s_ref[0] = s

    sq = jnp.sum(fb * fb, axis=0, keepdims=True)
    g = lax.dot_general(fc, fb, cdims, preferred_element_type=jnp.float32)
    score = sq - 2.0 * g
    iota = lax.broadcasted_iota(jnp.int32, score.shape, 1)
    boff = b * N
    inf = jnp.float32(jnp.inf)
    mprev = None
    for kstep in range(K):
        v = score if mprev is None else jnp.where(score > mprev, score, inf)
        i = iota
        w = N
        while w > 128:
            h = w // 2
            take = v[:, :h] <= v[:, h:w]
            v = jnp.where(take, v[:, :h], v[:, h:w])
            i = jnp.where(take, i[:, :h], i[:, h:w])
            w = h
        m = jnp.min(v, axis=1, keepdims=True)
        idx = jnp.min(jnp.where(v == m, i, N), axis=1, keepdims=True)
        knn_ref[0, :, kstep:kstep + 1] = idx + boff
        mprev = m


def _knn_conv(feature, w1t, w2t):
    grid = (B, N // T_KNN)
    return pl.pallas_call(
        _knn_conv_body,
        grid=grid,
        in_specs=[
            pl.BlockSpec((1, C, T_KNN), lambda b, t: (b, 0, t)),
            pl.BlockSpec((1, C, N), lambda b, t: (b, 0, 0)),
            pl.BlockSpec((C, C), lambda b, t: (0, 0)),
            pl.BlockSpec((C, C), lambda b, t: (0, 0)),
        ],
        out_specs=[
            pl.BlockSpec((1, T_KNN, K), lambda b, t: (b, t, 0)),
            pl.BlockSpec((1, T_KNN, C), lambda b, t: (b, t, 0)),
            pl.BlockSpec((1, T_KNN, CP), lambda b, t: (b, t, 0)),
        ],
        out_shape=[
            jax.ShapeDtypeStruct((B, N, K), jnp.int32),
            jax.ShapeDtypeStruct((B, N, C), jnp.float32),
            jax.ShapeDtypeStruct((B, N, CP), jnp.float32),
        ],
    )(feature, feature, w1t, w2t)


def _sc_gather(idx_flat, table):
    R = idx_flat.shape[0]
    D = table.shape[1]
    NW = 32
    Q = R // NW
    CH = 128
    nchunk = Q // CH
    mesh = plsc.VectorSubcoreMesh(core_axis_name="c", subcore_axis_name="s")

    @functools.partial(
        pl.kernel,
        mesh=mesh,
        out_type=jax.ShapeDtypeStruct((R, D), jnp.float32),
        scratch_types=[
            pltpu.VMEM((CH,), jnp.int32),
            pltpu.VMEM((CH, D), jnp.float32),
            pltpu.SemaphoreType.DMA,
        ],
    )
    def gk(idx_hbm, tab_hbm, out_hbm, idx_v, rows_v, sem):
        wid = lax.axis_index("s") * 2 + lax.axis_index("c")

        def chunk(i, carry):
            base = pl.multiple_of(wid * Q + i * CH, CH)
            pltpu.sync_copy(idx_hbm.at[pl.ds(base, CH)], idx_v)
            pltpu.async_copy(tab_hbm.at[idx_v], rows_v, sem).wait()
            pltpu.sync_copy(rows_v, out_hbm.at[pl.ds(base, CH)])
            return carry

        lax.fori_loop(0, nchunk, chunk, 0)

    return gk(idx_flat, table)


def _edge_stats_body(nb_ref, s_ref, mx_ref, mn_ref, sum_ref, ssq_ref):
    first = (pl.program_id(0) == 0) & (pl.program_id(1) == 0)
    s = s_ref[0]
    mx = None
    mn = None
    psum = jnp.zeros((1, C), jnp.float32)
    psq = jnp.zeros((1, C), jnp.float32)
    for k in range(K):
        ef = s - nb_ref[0, :, k, :C]
        mx = ef if mx is None else jnp.maximum(mx, ef)
        mn = ef if mn is None else jnp.minimum(mn, ef)
        psum = psum + jnp.sum(ef, axis=0, keepdims=True)
        psq = psq + jnp.sum(ef * ef, axis=0, keepdims=True)
    mx_ref[0] = mx
    mn_ref[0] = mn

    @pl.when(first)
    def _init():
        sum_ref[...] = jnp.zeros_like(sum_ref)
        ssq_ref[...] = jnp.zeros_like(ssq_ref)

    sum_ref[...] += psum
    ssq_ref[...] += psq


def _edge_stats(nb4, S):
    grid = (B, N // T_ST)
    return pl.pallas_call(
        _edge_stats_body,
        grid=grid,
        in_specs=[
            pl.BlockSpec((1, T_ST, K, CP), lambda b, t: (b, t, 0, 0)),
            pl.BlockSpec((1, T_ST, C), lambda b, t: (b, t, 0)),
        ],
        out_specs=[
            pl.BlockSpec((1, T_ST, C), lambda b, t: (b, t, 0)),
            pl.BlockSpec((1, T_ST, C), lambda b, t: (b, t, 0)),
            pl.BlockSpec((1, C), lambda b, t: (0, 0)),
            pl.BlockSpec((1, C), lambda b, t: (0, 0)),
        ],
        out_shape=[
            jax.ShapeDtypeStruct((B, N, C), jnp.float32),
            jax.ShapeDtypeStruct((B, N, C), jnp.float32),
            jax.ShapeDtypeStruct((1, C), jnp.float32),
            jax.ShapeDtypeStruct((1, C), jnp.float32),
        ],
    )(nb4, S)


def _finalize_body(mx_ref, mn_ref, sum_ref, ssq_ref, g_ref, be_ref, out_ref):
    cnt = float(B * N * K)
    mean = sum_ref[...] / cnt
    var = ssq_ref[...] / cnt - mean * mean
    inv = lax.rsqrt(var + 1e-5)
    scale = g_ref[...] * inv
    shift = be_ref[...] - mean * scale
    sel = jnp.where(scale >= 0.0, mx_ref[0], mn_ref[0])
    out_ref[0] = jnp.maximum(sel * scale + shift, 0.0).T


def _finalize(mx, mn, ssum, ssq, gamma2, beta2):
    grid = (B, N // T_FIN)
    return pl.pallas_call(
        _finalize_body,
        grid=grid,
        in_specs=[
            pl.BlockSpec((1, T_FIN, C), lambda b, t: (b, t, 0)),
            pl.BlockSpec((1, T_FIN, C), lambda b, t: (b, t, 0)),
            pl.BlockSpec((1, C), lambda b, t: (0, 0)),
            pl.BlockSpec((1, C), lambda b, t: (0, 0)),
            pl.BlockSpec((1, C), lambda b, t: (0, 0)),
            pl.BlockSpec((1, C), lambda b, t: (0, 0)),
        ],
        out_specs=pl.BlockSpec((1, C, T_FIN), lambda b, t: (b, 0, t)),
        out_shape=jax.ShapeDtypeStruct((B, C, N), jnp.float32),
    )(mx, mn, ssum, ssq, gamma2, beta2)


def kernel(feature, W1, W2, gamma, beta):
    knn, S, Et = _knn_conv(feature, W1.T, W2.T)
    nb_flat = _sc_gather(knn.reshape(-1), Et.reshape(B * N, CP))
    nb4 = nb_flat.reshape(B, N, K, CP)
    mx, mn, ssum, ssq = _edge_stats(nb4, S)
    return _finalize(mx, mn, ssum, ssq,
                     gamma.reshape(1, C), beta.reshape(1, C))


# speedup vs baseline: 18.7209x; 1.1279x over previous
import functools

import jax
import jax.numpy as jnp
from jax import lax
from jax.experimental import pallas as pl
from jax.experimental.pallas import tpu as pltpu
from jax.experimental.pallas import tpu_sc as plsc

B = 4
C = 64
N = 4096
K = 16

T_KNN = 256
T_ST = 128
T_FIN = 512
CP = 128


def _knn_conv_body(fc_ref, f_ref, w1t_ref, w2t_ref, knn_ref, s_ref, et_ref):
    fc = fc_ref[...]
    fb = f_ref[...]
    w1t = w1t_ref[...]
    w2t = w2t_ref[...]

    cdims = (((0,), (0,)), ((), ()))
    et = lax.dot_general(fc, w2t, cdims, preferred_element_type=jnp.float32)
    s = lax.dot_general(fc, w1t + w2t, cdims, preferred_element_type=jnp.float32)
    et_ref[:, :C] = et
    et_ref[:, C:] = jnp.zeros((fc.shape[1], CP - C), jnp.float32)
    s_ref[...] = s

    sq = jnp.sum(fb * fb, axis=0, keepdims=True)
    g = lax.dot_general(fc, fb, cdims, preferred_element_type=jnp.float32)
    score = sq - 2.0 * g
    iota = lax.broadcasted_iota(jnp.int32, score.shape, 1)
    inf = jnp.float32(jnp.inf)
    mprev = None
    for kstep in range(K):
        v = score if mprev is None else jnp.where(score > mprev, score, inf)
        i = iota
        w = N
        while w > 128:
            h = w // 2
            take = v[:, :h] <= v[:, h:w]
            v = jnp.where(take, v[:, :h], v[:, h:w])
            i = jnp.where(take, i[:, :h], i[:, h:w])
            w = h
        m = jnp.min(v, axis=1, keepdims=True)
        idx = jnp.min(jnp.where(v == m, i, N), axis=1, keepdims=True)
        knn_ref[:, kstep:kstep + 1] = idx
        mprev = m


def _knn_conv(fb, w1t, w2t):
    grid = (N // T_KNN,)
    return pl.pallas_call(
        _knn_conv_body,
        grid=grid,
        in_specs=[
            pl.BlockSpec((C, T_KNN), lambda t: (0, t)),
            pl.BlockSpec((C, N), lambda t: (0, 0)),
            pl.BlockSpec((C, C), lambda t: (0, 0)),
            pl.BlockSpec((C, C), lambda t: (0, 0)),
        ],
        out_specs=[
            pl.BlockSpec((T_KNN, K), lambda t: (t, 0)),
            pl.BlockSpec((T_KNN, C), lambda t: (t, 0)),
            pl.BlockSpec((T_KNN, CP), lambda t: (t, 0)),
        ],
        out_shape=[
            jax.ShapeDtypeStruct((N, K), jnp.int32),
            jax.ShapeDtypeStruct((N, C), jnp.float32),
            jax.ShapeDtypeStruct((N, CP), jnp.float32),
        ],
    )(fb, fb, w1t, w2t)


def _sc_gather(idx_flat, table):
    R = idx_flat.shape[0]
    D = table.shape[1]
    NW = 32
    Q = R // NW
    CH = 128
    nchunk = Q // CH
    mesh = plsc.VectorSubcoreMesh(core_axis_name="c", subcore_axis_name="s")

    @functools.partial(
        pl.kernel,
        mesh=mesh,
        out_type=jax.ShapeDtypeStruct((R, D), jnp.float32),
        scratch_types=[
            pltpu.VMEM((CH,), jnp.int32),
            pltpu.VMEM((CH, D), jnp.float32),
            pltpu.SemaphoreType.DMA,
        ],
    )
    def gk(idx_hbm, tab_hbm, out_hbm, idx_v, rows_v, sem):
        wid = lax.axis_index("s") * 2 + lax.axis_index("c")

        def chunk(i, carry):
            base = pl.multiple_of(wid * Q + i * CH, CH)
            pltpu.sync_copy(idx_hbm.at[pl.ds(base, CH)], idx_v)
            pltpu.async_copy(tab_hbm.at[idx_v], rows_v, sem).wait()
            pltpu.sync_copy(rows_v, out_hbm.at[pl.ds(base, CH)])
            return carry

        lax.fori_loop(0, nchunk, chunk, 0)

    return gk(idx_flat, table)


def _edge_stats_body(nb_ref, s_ref, mx_ref, mn_ref, sum_ref, ssq_ref):
    first = pl.program_id(0) == 0
    s = s_ref[...]
    mx = None
    mn = None
    psum = jnp.zeros((1, C), jnp.float32)
    psq = jnp.zeros((1, C), jnp.float32)
    for k in range(K):
        ef = s - nb_ref[:, k, :C]
        mx = ef if mx is None else jnp.maximum(mx, ef)
        mn = ef if mn is None else jnp.minimum(mn, ef)
        psum = psum + jnp.sum(ef, axis=0, keepdims=True)
        psq = psq + jnp.sum(ef * ef, axis=0, keepdims=True)
    mx_ref[...] = mx
    mn_ref[...] = mn

    @pl.when(first)
    def _init():
        sum_ref[...] = jnp.zeros_like(sum_ref)
        ssq_ref[...] = jnp.zeros_like(ssq_ref)

    sum_ref[...] += psum
    ssq_ref[...] += psq


def _edge_stats(nb3, S):
    grid = (N // T_ST,)
    return pl.pallas_call(
        _edge_stats_body,
        grid=grid,
        in_specs=[
            pl.BlockSpec((T_ST, K, CP), lambda t: (t, 0, 0)),
            pl.BlockSpec((T_ST, C), lambda t: (t, 0)),
        ],
        out_specs=[
            pl.BlockSpec((T_ST, C), lambda t: (t, 0)),
            pl.BlockSpec((T_ST, C), lambda t: (t, 0)),
            pl.BlockSpec((1, C), lambda t: (0, 0)),
            pl.BlockSpec((1, C), lambda t: (0, 0)),
        ],
        out_shape=[
            jax.ShapeDtypeStruct((N, C), jnp.float32),
            jax.ShapeDtypeStruct((N, C), jnp.float32),
            jax.ShapeDtypeStruct((1, C), jnp.float32),
            jax.ShapeDtypeStruct((1, C), jnp.float32),
        ],
    )(nb3, S)


def _finalize_body(mx_ref, mn_ref, sum_ref, ssq_ref, g_ref, be_ref, out_ref):
    cnt = float(B * N * K)
    mean = jnp.sum(sum_ref[...], axis=0, keepdims=True) / cnt
    var = jnp.sum(ssq_ref[...], axis=0, keepdims=True) / cnt - mean * mean
    inv = lax.rsqrt(var + 1e-5)
    scale = g_ref[...] * inv
    shift = be_ref[...] - mean * scale
    sel = jnp.where(scale >= 0.0, mx_ref[...], mn_ref[...])
    out_ref[...] = jnp.maximum(sel * scale + shift, 0.0).T


def _finalize(mx, mn, sums4, ssq4, gamma2, beta2):
    grid = (N // T_FIN,)
    return pl.pallas_call(
        _finalize_body,
        grid=grid,
        in_specs=[
            pl.BlockSpec((T_FIN, C), lambda t: (t, 0)),
            pl.BlockSpec((T_FIN, C), lambda t: (t, 0)),
            pl.BlockSpec((B, C), lambda t: (0, 0)),
            pl.BlockSpec((B, C), lambda t: (0, 0)),
            pl.BlockSpec((1, C), lambda t: (0, 0)),
            pl.BlockSpec((1, C), lambda t: (0, 0)),
        ],
        out_specs=pl.BlockSpec((C, T_FIN), lambda t: (0, t)),
        out_shape=jax.ShapeDtypeStruct((C, N), jnp.float32),
    )(mx, mn, sums4, ssq4, gamma2, beta2)


def kernel(feature, W1, W2, gamma, beta):
    w1t, w2t = W1.T, W2.T
    knns, Ss, Ets = [], [], []
    for b in range(B):
        knn_b, s_b, et_b = _knn_conv(feature[b], w1t, w2t)
        knns.append(knn_b)
        Ss.append(s_b)
        Ets.append(et_b)
    nbs = [_sc_gather(knns[b].reshape(-1), Ets[b]) for b in range(B)]
    mxs, mns, psums, pssqs = [], [], [], []
    for b in range(B):
        mx_b, mn_b, psum_b, pssq_b = _edge_stats(
            nbs[b].reshape(N, K, CP), Ss[b])
        mxs.append(mx_b)
        mns.append(mn_b)
        psums.append(psum_b)
        pssqs.append(pssq_b)
    sums4 = jnp.concatenate(psums, axis=0)
    ssq4 = jnp.concatenate(pssqs, axis=0)
    gamma2, beta2 = gamma.reshape(1, C), beta.reshape(1, C)
    outs = [_finalize(mxs[b], mns[b], sums4, ssq4, gamma2, beta2)
            for b in range(B)]
    return jnp.stack(outs, axis=0)


# speedup vs baseline: 18.7253x; 1.0002x over previous
import functools

import jax
import jax.numpy as jnp
from jax import lax
from jax.experimental import pallas as pl
from jax.experimental.pallas import tpu as pltpu
from jax.experimental.pallas import tpu_sc as plsc

B = 4
C = 64
N = 4096
K = 16

T_KNN = 256
T_ST = 128
T_FIN = 512
CP = 128


def _knn_conv_body(fc_ref, f_ref, w1t_ref, w2t_ref, knn_ref, s_ref, et_ref):
    fc = fc_ref[...]
    fb = f_ref[...]
    w1t = w1t_ref[...]
    w2t = w2t_ref[...]

    cdims = (((0,), (0,)), ((), ()))
    et = lax.dot_general(fc, w2t, cdims, preferred_element_type=jnp.float32)
    s = lax.dot_general(fc, w1t + w2t, cdims, preferred_element_type=jnp.float32)
    et_ref[:, :C] = et
    et_ref[:, C:] = jnp.zeros((fc.shape[1], CP - C), jnp.float32)
    s_ref[...] = s

    sq = jnp.sum(fb * fb, axis=0, keepdims=True)
    g = lax.dot_general(fc, fb, cdims, preferred_element_type=jnp.float32)
    score = sq - 2.0 * g
    iota = lax.broadcasted_iota(jnp.int32, score.shape, 1)
    inf = jnp.float32(jnp.inf)
    mprev = None
    for kstep in range(K):
        v = score if mprev is None else jnp.where(score > mprev, score, inf)
        i = iota
        w = N
        while w > 128:
            h = w // 2
            take = v[:, :h] <= v[:, h:w]
            v = jnp.where(take, v[:, :h], v[:, h:w])
            i = jnp.where(take, i[:, :h], i[:, h:w])
            w = h
        m = jnp.min(v, axis=1, keepdims=True)
        idx = jnp.min(jnp.where(v == m, i, N), axis=1, keepdims=True)
        knn_ref[:, kstep:kstep + 1] = idx
        mprev = m


def _knn_conv(fb, w1t, w2t):
    grid = (N // T_KNN,)
    return pl.pallas_call(
        _knn_conv_body,
        grid=grid,
        in_specs=[
            pl.BlockSpec((C, T_KNN), lambda t: (0, t)),
            pl.BlockSpec((C, N), lambda t: (0, 0)),
            pl.BlockSpec((C, C), lambda t: (0, 0)),
            pl.BlockSpec((C, C), lambda t: (0, 0)),
        ],
        out_specs=[
            pl.BlockSpec((T_KNN, K), lambda t: (t, 0)),
            pl.BlockSpec((T_KNN, C), lambda t: (t, 0)),
            pl.BlockSpec((T_KNN, CP), lambda t: (t, 0)),
        ],
        out_shape=[
            jax.ShapeDtypeStruct((N, K), jnp.int32),
            jax.ShapeDtypeStruct((N, C), jnp.float32),
            jax.ShapeDtypeStruct((N, CP), jnp.float32),
        ],
    )(fb, fb, w1t, w2t)


def _sc_gather(idx_flat, table):
    R = idx_flat.shape[0]
    D = table.shape[1]
    NW = 32
    Q = R // NW
    CH = 128
    nchunk = Q // CH
    mesh = plsc.VectorSubcoreMesh(core_axis_name="c", subcore_axis_name="s")

    @functools.partial(
        pl.kernel,
        mesh=mesh,
        out_type=jax.ShapeDtypeStruct((R, D), jnp.float32),
        scratch_types=[
            pltpu.VMEM((CH,), jnp.int32),
            pltpu.VMEM((CH, D), jnp.float32),
            pltpu.SemaphoreType.DMA,
        ],
    )
    def gk(idx_hbm, tab_hbm, out_hbm, idx_v, rows_v, sem):
        wid = lax.axis_index("s") * 2 + lax.axis_index("c")

        def chunk(i, carry):
            base = pl.multiple_of(wid * Q + i * CH, CH)
            pltpu.sync_copy(idx_hbm.at[pl.ds(base, CH)], idx_v)
            pltpu.async_copy(tab_hbm.at[idx_v], rows_v, sem).wait()
            pltpu.sync_copy(rows_v, out_hbm.at[pl.ds(base, CH)])
            return carry

        lax.fori_loop(0, nchunk, chunk, 0)

    return gk(idx_flat, table)


def _edge_stats_body(nb_ref, s_ref, mx_ref, mn_ref, sum_ref, ssq_ref):
    first = pl.program_id(0) == 0
    s = s_ref[...]
    ones = jnp.ones((1, s.shape[0]), jnp.float32)
    mx = None
    mn = None
    psum = jnp.zeros((1, C), jnp.float32)
    psq = jnp.zeros((1, C), jnp.float32)
    for k in range(K):
        ef = s - nb_ref[:, k, :C]
        mx = ef if mx is None else jnp.maximum(mx, ef)
        mn = ef if mn is None else jnp.minimum(mn, ef)
        psum = psum + jnp.dot(ones, ef, preferred_element_type=jnp.float32)
        psq = psq + jnp.dot(ones, ef * ef, preferred_element_type=jnp.float32)
    mx_ref[...] = mx
    mn_ref[...] = mn

    @pl.when(first)
    def _init():
        sum_ref[...] = jnp.zeros_like(sum_ref)
        ssq_ref[...] = jnp.zeros_like(ssq_ref)

    sum_ref[...] += psum
    ssq_ref[...] += psq


def _edge_stats(nb3, S):
    grid = (N // T_ST,)
    return pl.pallas_call(
        _edge_stats_body,
        grid=grid,
        in_specs=[
            pl.BlockSpec((T_ST, K, CP), lambda t: (t, 0, 0)),
            pl.BlockSpec((T_ST, C), lambda t: (t, 0)),
        ],
        out_specs=[
            pl.BlockSpec((T_ST, C), lambda t: (t, 0)),
            pl.BlockSpec((T_ST, C), lambda t: (t, 0)),
            pl.BlockSpec((1, C), lambda t: (0, 0)),
            pl.BlockSpec((1, C), lambda t: (0, 0)),
        ],
        out_shape=[
            jax.ShapeDtypeStruct((N, C), jnp.float32),
            jax.ShapeDtypeStruct((N, C), jnp.float32),
            jax.ShapeDtypeStruct((1, C), jnp.float32),
            jax.ShapeDtypeStruct((1, C), jnp.float32),
        ],
    )(nb3, S)


def _finalize_body(mx_ref, mn_ref, sum_ref, ssq_ref, g_ref, be_ref, out_ref):
    cnt = float(B * N * K)
    mean = jnp.sum(sum_ref[...], axis=0, keepdims=True) / cnt
    var = jnp.sum(ssq_ref[...], axis=0, keepdims=True) / cnt - mean * mean
    inv = lax.rsqrt(var + 1e-5)
    scale = g_ref[...] * inv
    shift = be_ref[...] - mean * scale
    sel = jnp.where(scale >= 0.0, mx_ref[...], mn_ref[...])
    out_ref[...] = jnp.maximum(sel * scale + shift, 0.0).T


def _finalize(mx, mn, sums4, ssq4, gamma2, beta2):
    grid = (N // T_FIN,)
    return pl.pallas_call(
        _finalize_body,
        grid=grid,
        in_specs=[
            pl.BlockSpec((T_FIN, C), lambda t: (t, 0)),
            pl.BlockSpec((T_FIN, C), lambda t: (t, 0)),
            pl.BlockSpec((B, C), lambda t: (0, 0)),
            pl.BlockSpec((B, C), lambda t: (0, 0)),
            pl.BlockSpec((1, C), lambda t: (0, 0)),
            pl.BlockSpec((1, C), lambda t: (0, 0)),
        ],
        out_specs=pl.BlockSpec((C, T_FIN), lambda t: (0, t)),
        out_shape=jax.ShapeDtypeStruct((C, N), jnp.float32),
    )(mx, mn, sums4, ssq4, gamma2, beta2)


def kernel(feature, W1, W2, gamma, beta):
    w1t, w2t = W1.T, W2.T
    knns, Ss, Ets = [], [], []
    for b in range(B):
        knn_b, s_b, et_b = _knn_conv(feature[b], w1t, w2t)
        knns.append(knn_b)
        Ss.append(s_b)
        Ets.append(et_b)
    nbs = [_sc_gather(knns[b].reshape(-1), Ets[b]) for b in range(B)]
    mxs, mns, psums, pssqs = [], [], [], []
    for b in range(B):
        mx_b, mn_b, psum_b, pssq_b = _edge_stats(
            nbs[b].reshape(N, K, CP), Ss[b])
        mxs.append(mx_b)
        mns.append(mn_b)
        psums.append(psum_b)
        pssqs.append(pssq_b)
    sums4 = jnp.concatenate(psums, axis=0)
    ssq4 = jnp.concatenate(pssqs, axis=0)
    gamma2, beta2 = gamma.reshape(1, C), beta.reshape(1, C)
    outs = [_finalize(mxs[b], mns[b], sums4, ssq4, gamma2, beta2)
            for b in range(B)]
    return jnp.stack(outs, axis=0)


# speedup vs baseline: 18.7413x; 1.0009x over previous
import functools

import jax
import jax.numpy as jnp
from jax import lax
from jax.experimental import pallas as pl
from jax.experimental.pallas import tpu as pltpu
from jax.experimental.pallas import tpu_sc as plsc

B = 4
C = 64
N = 4096
K = 16

T_KNN = 256
T_ST = 128
T_FIN = 512
CP = 128


def _knn_conv_body(fc_ref, f_ref, w1t_ref, w2t_ref, knn_ref, s_ref, et_ref):
    fc = fc_ref[...]
    fb = f_ref[...]
    w1t = w1t_ref[...]
    w2t = w2t_ref[...]

    cdims = (((0,), (0,)), ((), ()))
    et = lax.dot_general(fc, w2t, cdims, preferred_element_type=jnp.float32)
    s = lax.dot_general(fc, w1t + w2t, cdims, preferred_element_type=jnp.float32)
    et_ref[:, :C] = et
    et_ref[:, C:] = jnp.zeros((fc.shape[1], CP - C), jnp.float32)
    s_ref[...] = s

    sq = jnp.sum(fb * fb, axis=0, keepdims=True)
    g = lax.dot_general(fc, fb, cdims, preferred_element_type=jnp.float32)
    score = sq - 2.0 * g
    iota = lax.broadcasted_iota(jnp.int32, score.shape, 1)
    inf = jnp.float32(jnp.inf)
    mprev = None
    for kstep in range(K):
        v = score if mprev is None else jnp.where(score > mprev, score, inf)
        i = iota
        w = N
        while w > 128:
            h = w // 2
            take = v[:, :h] <= v[:, h:w]
            v = jnp.where(take, v[:, :h], v[:, h:w])
            i = jnp.where(take, i[:, :h], i[:, h:w])
            w = h
        m = jnp.min(v, axis=1, keepdims=True)
        idx = jnp.min(jnp.where(v == m, i, N), axis=1, keepdims=True)
        knn_ref[:, kstep:kstep + 1] = idx
        mprev = m


def _knn_conv(fb, w1t, w2t):
    grid = (N // T_KNN,)
    return pl.pallas_call(
        _knn_conv_body,
        grid=grid,
        in_specs=[
            pl.BlockSpec((C, T_KNN), lambda t: (0, t)),
            pl.BlockSpec((C, N), lambda t: (0, 0)),
            pl.BlockSpec((C, C), lambda t: (0, 0)),
            pl.BlockSpec((C, C), lambda t: (0, 0)),
        ],
        out_specs=[
            pl.BlockSpec((T_KNN, K), lambda t: (t, 0)),
            pl.BlockSpec((T_KNN, C), lambda t: (t, 0)),
            pl.BlockSpec((T_KNN, CP), lambda t: (t, 0)),
        ],
        out_shape=[
            jax.ShapeDtypeStruct((N, K), jnp.int32),
            jax.ShapeDtypeStruct((N, C), jnp.float32),
            jax.ShapeDtypeStruct((N, CP), jnp.float32),
        ],
    )(fb, fb, w1t, w2t)


def _sc_gather(knn, table):
    R = N * K
    D = table.shape[1]
    NW = 32
    RW = N // NW
    RC = 8
    CH = RC * K
    nchunk = RW // RC
    mesh = plsc.VectorSubcoreMesh(core_axis_name="c", subcore_axis_name="s")

    @functools.partial(
        pl.kernel,
        mesh=mesh,
        out_type=jax.ShapeDtypeStruct((R, D), jnp.float32),
        scratch_types=[
            pltpu.VMEM((RC, K), jnp.int32),
            pltpu.VMEM((CH,), jnp.int32),
            pltpu.VMEM((CH, D), jnp.float32),
            pltpu.SemaphoreType.DMA,
        ],
    )
    def gk(idx_hbm, tab_hbm, out_hbm, idx2_v, idx_v, rows_v, sem):
        wid = lax.axis_index("s") * 2 + lax.axis_index("c")

        def chunk(i, carry):
            rb = pl.multiple_of(wid * RW + i * RC, RC)
            pltpu.sync_copy(idx_hbm.at[pl.ds(rb, RC), :], idx2_v)
            for r in range(RC):
                idx_v[pl.ds(r * K, K)] = idx2_v[r, :]
            pltpu.async_copy(tab_hbm.at[idx_v], rows_v, sem).wait()
            pltpu.sync_copy(rows_v, out_hbm.at[pl.ds(rb * K, CH)])
            return carry

        lax.fori_loop(0, nchunk, chunk, 0)

    return gk(knn, table)


def _edge_stats_body(nb_ref, s_ref, mx_ref, mn_ref, sum_ref, ssq_ref):
    first = pl.program_id(0) == 0
    s = s_ref[...]
    ones = jnp.ones((1, s.shape[0]), jnp.float32)
    mx = None
    mn = None
    psum = jnp.zeros((1, C), jnp.float32)
    psq = jnp.zeros((1, C), jnp.float32)
    for k in range(K):
        ef = s - nb_ref[:, k, :C]
        mx = ef if mx is None else jnp.maximum(mx, ef)
        mn = ef if mn is None else jnp.minimum(mn, ef)
        psum = psum + jnp.dot(ones, ef, preferred_element_type=jnp.float32)
        psq = psq + jnp.dot(ones, ef * ef, preferred_element_type=jnp.float32)
    mx_ref[...] = mx
    mn_ref[...] = mn

    @pl.when(first)
    def _init():
        sum_ref[...] = jnp.zeros_like(sum_ref)
        ssq_ref[...] = jnp.zeros_like(ssq_ref)

    sum_ref[...] += psum
    ssq_ref[...] += psq


def _edge_stats(nb3, S):
    grid = (N // T_ST,)
    return pl.pallas_call(
        _edge_stats_body,
        grid=grid,
        in_specs=[
            pl.BlockSpec((T_ST, K, CP), lambda t: (t, 0, 0)),
            pl.BlockSpec((T_ST, C), lambda t: (t, 0)),
        ],
        out_specs=[
            pl.BlockSpec((T_ST, C), lambda t: (t, 0)),
            pl.BlockSpec((T_ST, C), lambda t: (t, 0)),
            pl.BlockSpec((1, C), lambda t: (0, 0)),
            pl.BlockSpec((1, C), lambda t: (0, 0)),
        ],
        out_shape=[
            jax.ShapeDtypeStruct((N, C), jnp.float32),
            jax.ShapeDtypeStruct((N, C), jnp.float32),
            jax.ShapeDtypeStruct((1, C), jnp.float32),
            jax.ShapeDtypeStruct((1, C), jnp.float32),
        ],
    )(nb3, S)


def _finalize_body(mx0, mx1, mx2, mx3, mn0, mn1, mn2, mn3,
                   sum_ref, ssq_ref, g_ref, be_ref, out_ref):
    b = pl.program_id(0)
    cnt = float(B * N * K)
    mean = jnp.sum(sum_ref[...], axis=0, keepdims=True) / cnt
    var = jnp.sum(ssq_ref[...], axis=0, keepdims=True) / cnt - mean * mean
    inv = lax.rsqrt(var + 1e-5)
    scale = g_ref[...] * inv
    shift = be_ref[...] - mean * scale
    mx = jnp.where(b == 0, mx0[...],
                   jnp.where(b == 1, mx1[...],
                             jnp.where(b == 2, mx2[...], mx3[...])))
    mn = jnp.where(b == 0, mn0[...],
                   jnp.where(b == 1, mn1[...],
                             jnp.where(b == 2, mn2[...], mn3[...])))
    sel = jnp.where(scale >= 0.0, mx, mn)
    out_ref[0] = jnp.maximum(sel * scale + shift, 0.0).T


def _finalize(mxs, mns, sums4, ssq4, gamma2, beta2):
    grid = (B, N // T_FIN)
    tile = pl.BlockSpec((T_FIN, C), lambda b, t: (t, 0))
    small = pl.BlockSpec((B, C), lambda b, t: (0, 0))
    vec = pl.BlockSpec((1, C), lambda b, t: (0, 0))
    return pl.pallas_call(
        _finalize_body,
        grid=grid,
        in_specs=[tile] * 8 + [small, small, vec, vec],
        out_specs=pl.BlockSpec((1, C, T_FIN), lambda b, t: (b, 0, t)),
        out_shape=jax.ShapeDtypeStruct((B, C, N), jnp.float32),
    )(*mxs, *mns, sums4, ssq4, gamma2, beta2)


def kernel(feature, W1, W2, gamma, beta):
    w1t, w2t = W1.T, W2.T
    knns, Ss, Ets = [], [], []
    for b in range(B):
        knn_b, s_b, et_b = _knn_conv(feature[b], w1t, w2t)
        knns.append(knn_b)
        Ss.append(s_b)
        Ets.append(et_b)
    nbs = [_sc_gather(knns[b], Ets[b]) for b in range(B)]
    mxs, mns, psums, pssqs = [], [], [], []
    for b in range(B):
        mx_b, mn_b, psum_b, pssq_b = _edge_stats(
            nbs[b].reshape(N, K, CP), Ss[b])
        mxs.append(mx_b)
        mns.append(mn_b)
        psums.append(psum_b)
        pssqs.append(pssq_b)
    sums4 = jnp.concatenate(psums, axis=0)
    ssq4 = jnp.concatenate(pssqs, axis=0)
    gamma2, beta2 = gamma.reshape(1, C), beta.reshape(1, C)
    return _finalize(mxs, mns, sums4, ssq4, gamma2, beta2)
